# Initial kernel scaffold; baseline (speedup 1.0000x reference)
import jax
import jax.numpy as jnp
from jax.experimental import pallas as pl


def kernel(X, pos_edges, neg_edges, target, pos_surr, neg_surr, pos_base_w, pos_base_b, neg_base_w, neg_base_b, pos_deep_w, pos_deep_b, neg_deep_w, neg_deep_b, regression_weights, fc_weights):
    raise NotImplementedError("write your pallas kernel here")


# speedup vs baseline: 1.0081x; 1.0081x over previous
import functools

import jax
import jax.numpy as jnp
from jax.experimental import pallas as pl
from jax.experimental.pallas import tpu as pltpu

_ROW_TILE = 2048
_EDGE_TILE = 512


def _round_up(x, m):
    return ((x + m - 1) // m) * m


def _row_tile(n, cap=_ROW_TILE):
    t = min(cap, n)
    while t > 128 and n % t:
        t -= 128
    return t


def _proj_kernel(x_ref, wy_ref, ws_ref, b_ref, y_ref, s_ref):
    x = x_ref[...].astype(jnp.bfloat16)
    y_ref[...] = jnp.dot(x, wy_ref[...],
                         preferred_element_type=jnp.float32).astype(jnp.bfloat16)
    s_ref[...] = (jnp.dot(x, ws_ref[...], preferred_element_type=jnp.float32)
                  + b_ref[...])


def _project(x, wy, ws, b):
    n, f = x.shape
    tm = _row_tile(n, 1024)
    wy = wy.astype(jnp.bfloat16)
    ws = ws.astype(jnp.bfloat16)
    return pl.pallas_call(
        _proj_kernel,
        out_shape=(jax.ShapeDtypeStruct((n, wy.shape[1]), jnp.bfloat16),
                   jax.ShapeDtypeStruct((n, ws.shape[1]), jnp.float32)),
        grid=(n // tm,),
        in_specs=[
            pl.BlockSpec((tm, f), lambda i: (i, 0)),
            pl.BlockSpec(wy.shape, lambda i: (0, 0)),
            pl.BlockSpec(ws.shape, lambda i: (0, 0)),
            pl.BlockSpec(b.shape, lambda i: (0, 0)),
        ],
        out_specs=(pl.BlockSpec((tm, wy.shape[1]), lambda i: (i, 0)),
                   pl.BlockSpec((tm, ws.shape[1]), lambda i: (i, 0))),
        compiler_params=pltpu.CompilerParams(
            dimension_semantics=("parallel",)),
    )(x, wy, ws, b)


def _norm_tanh(hp, hn, dtype):
    inv_p = jax.lax.rsqrt(
        jnp.maximum(jnp.sum(hp * hp, axis=-1, keepdims=True), 1e-24))
    inv_n = jax.lax.rsqrt(
        jnp.maximum(jnp.sum(hn * hn, axis=-1, keepdims=True), 1e-24))
    return jnp.tanh(
        jnp.concatenate([hp * inv_p, hn * inv_n], axis=-1)).astype(dtype)


def _sage_base_kernel(cp_ref, cn_ref, y_ref, s_ref, sp_ref, sn_ref,
                      o_ref, accp_ref, accn_ref):
    k = pl.program_id(1)

    @pl.when(k == 0)
    def _():
        accp_ref[...] = jnp.zeros_like(accp_ref)
        accn_ref[...] = jnp.zeros_like(accn_ref)

    y = y_ref[...]
    h = accp_ref.shape[1]
    accp_ref[...] += jnp.dot(cp_ref[...].astype(jnp.bfloat16), y[:, :h],
                             preferred_element_type=jnp.float32)
    accn_ref[...] += jnp.dot(cn_ref[...].astype(jnp.bfloat16), y[:, h:],
                             preferred_element_type=jnp.float32)

    @pl.when(k == pl.num_programs(1) - 1)
    def _():
        s = s_ref[...]
        hp = accp_ref[...] * sp_ref[...] + s[:, :h]
        hn = accn_ref[...] * sn_ref[...] + s[:, h:]
        o_ref[...] = _norm_tanh(hp, hn, o_ref.dtype)


def _sage_deep_kernel(cp_ref, cn_ref, y_ref, s_ref, sp_ref, sn_ref,
                      o32_ref, obf_ref, accp_ref, accn_ref):
    i = pl.program_id(0)
    k = pl.program_id(1)

    @pl.when(k == 0)
    def _():
        accp_ref[...] = jnp.zeros_like(accp_ref)
        accn_ref[...] = jnp.zeros_like(accn_ref)

    y = y_ref[...]
    h2 = accp_ref.shape[1]
    y1, y2 = y[:, :h2], y[:, h2:]
    accp_ref[...] += jnp.dot(cp_ref[...].astype(jnp.bfloat16), y1,
                             preferred_element_type=jnp.float32)
    accn_ref[...] += jnp.dot(cn_ref[...].astype(jnp.bfloat16), y2,
                             preferred_element_type=jnp.float32)

    @pl.when(k == i)
    def _():
        accp_ref[...] += y1.astype(jnp.float32)
        accn_ref[...] += y2.astype(jnp.float32)

    @pl.when(k == pl.num_programs(1) - 1)
    def _():
        h = (accp_ref[...] * sp_ref[...] + accn_ref[...] * sn_ref[...]
             + s_ref[...])
        hh = h.shape[1] // 2
        z = _norm_tanh(h[:, :hh], h[:, hh:], jnp.float32)
        o32_ref[...] = z
        obf_ref[...] = z.astype(jnp.bfloat16)


def _sage_layer(cp, cn, y, s, sp, sn, deep):
    n = cp.shape[0]
    tm = tk = _row_tile(n)
    h2 = s.shape[1]
    hacc = y.shape[1] // 2
    grid = (n // tm, n // tk)
    in_specs = [
        pl.BlockSpec((tm, tk), lambda i, k: (i, k)),
        pl.BlockSpec((tm, tk), lambda i, k: (i, k)),
        pl.BlockSpec((tk, 2 * hacc), lambda i, k: (k, 0)),
        pl.BlockSpec((tm, h2), lambda i, k: (i, 0)),
        pl.BlockSpec((tm, 1), lambda i, k: (i, 0)),
        pl.BlockSpec((tm, 1), lambda i, k: (i, 0)),
    ]
    scratch = [pltpu.VMEM((tm, hacc), jnp.float32),
               pltpu.VMEM((tm, hacc), jnp.float32)]
    params = pltpu.CompilerParams(
        dimension_semantics=("parallel", "arbitrary"),
        vmem_limit_bytes=56 * 1024 * 1024,
    )
    if deep:
        return pl.pallas_call(
            _sage_deep_kernel,
            out_shape=(jax.ShapeDtypeStruct((n, h2), jnp.float32),
                       jax.ShapeDtypeStruct((n, h2), jnp.bfloat16)),
            grid_spec=pltpu.PrefetchScalarGridSpec(
                num_scalar_prefetch=0, grid=grid, in_specs=in_specs,
                out_specs=(pl.BlockSpec((tm, h2), lambda i, k: (i, 0)),
                           pl.BlockSpec((tm, h2), lambda i, k: (i, 0))),
                scratch_shapes=scratch),
            compiler_params=params,
        )(cp, cn, y, s, sp, sn)
    return pl.pallas_call(
        _sage_base_kernel,
        out_shape=jax.ShapeDtypeStruct((n, h2), jnp.bfloat16),
        grid_spec=pltpu.PrefetchScalarGridSpec(
            num_scalar_prefetch=0, grid=grid, in_specs=in_specs,
            out_specs=pl.BlockSpec((tm, h2), lambda i, k: (i, 0)),
            scratch_shapes=scratch),
        compiler_params=params,
    )(cp, cn, y, s, sp, sn)


def _loss_kernel(pzi_ref, pzj_ref, pzk_ref, nzi_ref, nzj_ref, nzk_ref,
                 t0_ref, t1_ref, t2_ref, t3_ref, t4_ref, t5_ref,
                 wab_ref, wfc_ref,
                 hp_ref, hn_ref, nl_ref,
                 p0_ref, p1_ref, p2_ref, p3_ref, p4_ref, p5_ref,
                 hps_ref, hns_ref, nls_ref, *, n_pos, n_neg, eb):
    c = pl.program_id(0)
    e = pl.program_id(1)

    @pl.when(e == 0)
    def _():
        hps_ref[...] = jnp.zeros_like(hps_ref)
        hns_ref[...] = jnp.zeros_like(hns_ref)
        nls_ref[...] = jnp.zeros_like(nls_ref)

    te = pzi_ref.shape[0]
    rows = (c * eb + e) * te + jax.lax.broadcasted_iota(jnp.int32, (te, 1), 0)
    pmask = rows < n_pos
    nmask = rows < n_neg

    blocks = [pzi_ref[...], pzj_ref[...], pzk_ref[...],
              nzi_ref[...], nzj_ref[...], nzk_ref[...]]
    pzi, pzj, pzk, nzi, nzj, nzk = [b.astype(jnp.float32) for b in blocks]

    d_pij = jnp.sum((pzi - pzj) ** 2, axis=-1, keepdims=True)
    d_pik = jnp.sum((pzi - pzk) ** 2, axis=-1, keepdims=True)
    hps_ref[...] += jnp.sum(jnp.where(pmask, jnp.maximum(d_pij - d_pik, 0.0), 0.0))
    d_nij = jnp.sum((nzi - nzj) ** 2, axis=-1, keepdims=True)
    d_nik = jnp.sum((nzi - nzk) ** 2, axis=-1, keepdims=True)
    hns_ref[...] += jnp.sum(jnp.where(nmask, jnp.maximum(d_nik - d_nij, 0.0), 0.0))

    stacked = jnp.concatenate(blocks, axis=0)
    proj = jnp.dot(stacked, wab_ref[...], preferred_element_type=jnp.float32)
    h = proj.shape[1] // 2

    def a_of(g):
        return proj[g * te:(g + 1) * te, :h]

    def b_of(g):
        return proj[g * te:(g + 1) * te, h:]

    pairs = [(0, 1), (3, 4), (3, 5), (4, 5), (0, 2), (1, 2)]
    p_all = jnp.concatenate([a_of(a) + b_of(b) for a, b in pairs], axis=0)
    logits_all = jnp.dot(p_all, wfc_ref[...],
                         preferred_element_type=jnp.float32)

    tgts = [t0_ref[...], t1_ref[...], t2_ref[...], t3_ref[...],
            t4_ref[...], t5_ref[...]]
    masks = [pmask, nmask, nmask, nmask, pmask, pmask]
    preds = [p0_ref, p1_ref, p2_ref, p3_ref, p4_ref, p5_ref]
    nll_sum = 0.0
    for g in range(6):
        logits = logits_all[g * te:(g + 1) * te, :]
        mx = jnp.max(logits, axis=-1, keepdims=True)
        lse = mx + jnp.log(jnp.sum(jnp.exp(logits - mx), axis=-1, keepdims=True))
        logsm = logits - lse
        preds[g][...] = logsm
        classes = jax.lax.broadcasted_iota(jnp.int32, logsm.shape, 1)
        onehot = (classes == tgts[g]).astype(jnp.float32)
        nll = -jnp.sum(onehot * logsm, axis=-1, keepdims=True)
        nll_sum = nll_sum + jnp.sum(jnp.where(masks[g], nll, 0.0))
    nls_ref[...] += nll_sum

    @pl.when(e == eb - 1)
    def _():
        hp_ref[...] = hps_ref[...][None]
        hn_ref[...] = hns_ref[...][None]
        nl_ref[...] = nls_ref[...][None]


def _fused_loss_opt(pz_i, pz_j, pz_k, nz_i, nz_j, nz_k, w_reg, w_fc,
                    target, lamb):
    e_pos, two_h = pz_i.shape
    e_neg = nz_i.shape[0]
    m_rows = 3 * (e_pos + e_neg)
    e_max = max(e_pos, e_neg)
    te = _EDGE_TILE if e_max > _EDGE_TILE else max(_round_up(e_max, 8), 8)
    e_pad = _round_up(e_max, te)
    nblk = e_pad // te
    ncore = 2 if nblk % 2 == 0 else 1
    eb = nblk // ncore

    def pad_rows(a):
        return jnp.pad(a, ((0, e_pad - a.shape[0]), (0, 0)))

    zs = [pad_rows(a) for a in (pz_i, pz_j, pz_k, nz_i, nz_j, nz_k)]

    sizes = [e_pos, e_neg, e_neg, e_neg, e_pos, e_pos]
    offs = [0]
    for sz in sizes[:-1]:
        offs.append(offs[-1] + sz)
    tgt = target.astype(jnp.int32)
    tgts = [jnp.pad(tgt[o:o + sz], (0, e_pad - sz)).reshape(e_pad, 1)
            for o, sz in zip(offs, sizes)]

    h = w_reg.shape[1]
    wab = jnp.concatenate([w_reg[:two_h], w_reg[two_h:]],
                          axis=1).astype(jnp.bfloat16)

    kern = functools.partial(_loss_kernel, n_pos=e_pos, n_neg=e_neg, eb=eb)
    edge_spec = pl.BlockSpec((te, two_h), lambda c, e, eb=eb: (c * eb + e, 0))
    tgt_spec = pl.BlockSpec((te, 1), lambda c, e, eb=eb: (c * eb + e, 0))
    pred_spec = pl.BlockSpec((te, 3), lambda c, e, eb=eb: (c * eb + e, 0))
    part_spec = pl.BlockSpec((1, 1, 1), lambda c, e: (c, 0, 0))

    outs = pl.pallas_call(
        kern,
        out_shape=(tuple(jax.ShapeDtypeStruct((ncore, 1, 1), jnp.float32)
                         for _ in range(3))
                   + tuple(jax.ShapeDtypeStruct((e_pad, 3), jnp.float32)
                           for _ in range(6))),
        grid_spec=pltpu.PrefetchScalarGridSpec(
            num_scalar_prefetch=0,
            grid=(ncore, eb),
            in_specs=[edge_spec] * 6 + [tgt_spec] * 6
                     + [pl.BlockSpec((two_h, two_h), lambda c, e: (0, 0)),
                        pl.BlockSpec((h, 3), lambda c, e: (0, 0))],
            out_specs=(part_spec,) * 3 + (pred_spec,) * 6,
            scratch_shapes=[pltpu.VMEM((1, 1), jnp.float32)] * 3),
        compiler_params=pltpu.CompilerParams(
            dimension_semantics=("parallel", "arbitrary")),
    )(*zs, *tgts, wab, w_fc)

    hp_sum = jnp.sum(outs[0])
    hn_sum = jnp.sum(outs[1])
    nl_sum = jnp.sum(outs[2])
    loss = (nl_sum * (1.0 / m_rows)
            + lamb * (hp_sum * (1.0 / e_pos) + hn_sum * (1.0 / e_neg)))
    preds = outs[3:]
    predictions_soft = jnp.concatenate(
        [p[:sz] for p, sz in zip(preds, sizes)], axis=0)
    return loss, predictions_soft


def _adj_counts(edges, n):
    row, col = edges[0], edges[1]
    w = row != col
    cnt = jnp.zeros((n, n), jnp.int8).at[row, col].add(w.astype(jnp.int8))
    deg = jnp.zeros((n, 1), jnp.float32).at[row, 0].add(w.astype(jnp.float32))
    return cnt, deg


def kernel(X, pos_edges, neg_edges, target, pos_surr, neg_surr,
           pos_base_w, pos_base_b, neg_base_w, neg_base_b,
           pos_deep_w, pos_deep_b, neg_deep_w, neg_deep_b,
           regression_weights, fc_weights):
    n, f = X.shape
    h1 = pos_base_w.shape[1]
    h2d = pos_deep_w.shape[1]

    cp, dp = _adj_counts(pos_edges, n)
    cn, dn = _adj_counts(neg_edges, n)
    sp1 = jnp.where(dp > 0, 1.0 / jnp.maximum(dp, 1.0), 0.0)
    sn1 = jnp.where(dn > 0, 1.0 / jnp.maximum(dn, 1.0), 0.0)
    sp2 = 1.0 / (dp + 1.0)
    sn2 = 1.0 / (dn + 1.0)

    wy1 = jnp.concatenate([pos_base_w[:f], neg_base_w[:f]], axis=1)
    ws1 = jnp.concatenate([pos_base_w[f:], neg_base_w[f:]], axis=1)
    b1 = jnp.concatenate([pos_base_b, neg_base_b], axis=1)
    y1, s1 = _project(X, wy1, ws1, b1)
    hc = _sage_layer(cp, cn, y1, s1, sp1, sn1, deep=False)

    wp, wn = pos_deep_w, neg_deep_w
    z128 = jnp.zeros((h2d, h2d), jnp.float32)
    w1d = jnp.concatenate(
        [jnp.concatenate([wp[0:h1], z128], axis=1),
         jnp.concatenate([z128, wn[0:h1]], axis=1)], axis=0)
    w2d = jnp.concatenate(
        [jnp.concatenate([z128, wn[h1:2 * h1]], axis=1),
         jnp.concatenate([wp[h1:2 * h1], z128], axis=1)], axis=0)
    w3d = jnp.concatenate(
        [jnp.concatenate([wp[2 * h1:3 * h1], wn[3 * h1:4 * h1]], axis=1),
         jnp.concatenate([wp[3 * h1:4 * h1], wn[2 * h1:3 * h1]], axis=1)],
        axis=0)
    b2 = jnp.concatenate([pos_deep_b, neg_deep_b], axis=1)
    y2, s2 = _project(hc, jnp.concatenate([w1d, w2d], axis=1), w3d, b2)
    z, zb = _sage_layer(cp, cn, y2, s2, sp2, sn2, deep=True)

    pz_i = jnp.take(zb, pos_edges[0], axis=0)
    pz_j = jnp.take(zb, pos_edges[1], axis=0)
    pz_k = jnp.take(zb, pos_surr, axis=0)
    nz_i = jnp.take(zb, neg_edges[0], axis=0)
    nz_j = jnp.take(zb, neg_edges[1], axis=0)
    nz_k = jnp.take(zb, neg_surr, axis=0)

    loss, predictions_soft = _fused_loss_opt(
        pz_i, pz_j, pz_k, nz_i, nz_j, nz_k,
        regression_weights, fc_weights, target, 1.0)
    return loss, z, predictions_soft


# speedup vs baseline: 1.0483x; 1.0398x over previous
import functools

import jax
import jax.numpy as jnp
from jax.experimental import pallas as pl
from jax.experimental.pallas import tpu as pltpu

_ROW_TILE = 2048
_EDGE_TILE = 512


def _round_up(x, m):
    return ((x + m - 1) // m) * m


def _row_tile(n, cap=_ROW_TILE):
    t = min(cap, n)
    while t > 128 and n % t:
        t -= 128
    return t


def _proj_kernel(x_ref, wy_ref, ws_ref, b_ref, y_ref, s_ref, *, ones_split):
    x = x_ref[...].astype(jnp.bfloat16)
    y = jnp.dot(x, wy_ref[...], preferred_element_type=jnp.float32)
    if ones_split:
        h = y.shape[1] // 2
        lane = jax.lax.broadcasted_iota(jnp.int32, (y.shape[0], h), 1)
        e = (lane == 0).astype(jnp.bfloat16)
        yb = y.astype(jnp.bfloat16)
        y_ref[...] = jnp.concatenate([yb[:, :h], e, yb[:, h:], e], axis=1)
    else:
        y_ref[...] = y.astype(jnp.bfloat16)
    s_ref[...] = (jnp.dot(x, ws_ref[...], preferred_element_type=jnp.float32)
                  + b_ref[...])


def _project(x, wy, ws, b, ones_split=False):
    n, f = x.shape
    tm = _row_tile(n, 1024)
    wy = wy.astype(jnp.bfloat16)
    ws = ws.astype(jnp.bfloat16)
    y_cols = 2 * wy.shape[1] if ones_split else wy.shape[1]
    return pl.pallas_call(
        functools.partial(_proj_kernel, ones_split=ones_split),
        out_shape=(jax.ShapeDtypeStruct((n, y_cols), jnp.bfloat16),
                   jax.ShapeDtypeStruct((n, ws.shape[1]), jnp.float32)),
        grid=(n // tm,),
        in_specs=[
            pl.BlockSpec((tm, f), lambda i: (i, 0)),
            pl.BlockSpec(wy.shape, lambda i: (0, 0)),
            pl.BlockSpec(ws.shape, lambda i: (0, 0)),
            pl.BlockSpec(b.shape, lambda i: (0, 0)),
        ],
        out_specs=(pl.BlockSpec((tm, y_cols), lambda i: (i, 0)),
                   pl.BlockSpec((tm, ws.shape[1]), lambda i: (i, 0))),
        compiler_params=pltpu.CompilerParams(
            dimension_semantics=("parallel",)),
    )(x, wy, ws, b)


def _norm_tanh(hp, hn, dtype):
    inv_p = jax.lax.rsqrt(
        jnp.maximum(jnp.sum(hp * hp, axis=-1, keepdims=True), 1e-24))
    inv_n = jax.lax.rsqrt(
        jnp.maximum(jnp.sum(hn * hn, axis=-1, keepdims=True), 1e-24))
    return jnp.tanh(
        jnp.concatenate([hp * inv_p, hn * inv_n], axis=-1)).astype(dtype)


def _sage_base_kernel(cp_ref, cn_ref, y_ref, s_ref,
                      o_ref, dp_ref, dn_ref, accp_ref, accn_ref):
    k = pl.program_id(1)

    @pl.when(k == 0)
    def _():
        accp_ref[...] = jnp.zeros_like(accp_ref)
        accn_ref[...] = jnp.zeros_like(accn_ref)

    y = y_ref[...]
    w = accp_ref.shape[1]
    accp_ref[...] += jnp.dot(cp_ref[...].astype(jnp.bfloat16), y[:, :w],
                             preferred_element_type=jnp.float32)
    accn_ref[...] += jnp.dot(cn_ref[...].astype(jnp.bfloat16), y[:, w:],
                             preferred_element_type=jnp.float32)

    @pl.when(k == pl.num_programs(1) - 1)
    def _():
        h = w // 2
        s = s_ref[...]
        accp = accp_ref[...]
        accn = accn_ref[...]
        dp = accp[:, h:h + 1]
        dn = accn[:, h:h + 1]
        sp = jnp.where(dp > 0, 1.0 / jnp.maximum(dp, 1.0), 0.0)
        sn = jnp.where(dn > 0, 1.0 / jnp.maximum(dn, 1.0), 0.0)
        hp = accp[:, :h] * sp + s[:, :h]
        hn = accn[:, :h] * sn + s[:, h:]
        o_ref[...] = _norm_tanh(hp, hn, o_ref.dtype)
        dp_ref[...] = dp
        dn_ref[...] = dn


def _sage_deep_kernel(cp_ref, cn_ref, y_ref, s_ref, sp_ref, sn_ref,
                      o32_ref, obf_ref, accp_ref, accn_ref):
    i = pl.program_id(0)
    k = pl.program_id(1)

    @pl.when(k == 0)
    def _():
        accp_ref[...] = jnp.zeros_like(accp_ref)
        accn_ref[...] = jnp.zeros_like(accn_ref)

    y = y_ref[...]
    h2 = accp_ref.shape[1]
    y1, y2 = y[:, :h2], y[:, h2:]
    accp_ref[...] += jnp.dot(cp_ref[...].astype(jnp.bfloat16), y1,
                             preferred_element_type=jnp.float32)
    accn_ref[...] += jnp.dot(cn_ref[...].astype(jnp.bfloat16), y2,
                             preferred_element_type=jnp.float32)

    @pl.when(k == i)
    def _():
        accp_ref[...] += y1.astype(jnp.float32)
        accn_ref[...] += y2.astype(jnp.float32)

    @pl.when(k == pl.num_programs(1) - 1)
    def _():
        h = (accp_ref[...] * sp_ref[...] + accn_ref[...] * sn_ref[...]
             + s_ref[...])
        hh = h.shape[1] // 2
        z = _norm_tanh(h[:, :hh], h[:, hh:], jnp.float32)
        o32_ref[...] = z
        obf_ref[...] = z.astype(jnp.bfloat16)


def _sage_layer(cp, cn, y, s, sp, sn, deep):
    n = cp.shape[0]
    tm = tk = _row_tile(n)
    h2 = s.shape[1]
    hacc = y.shape[1] // 2
    grid = (n // tm, n // tk)
    in_specs = [
        pl.BlockSpec((tm, tk), lambda i, k: (i, k)),
        pl.BlockSpec((tm, tk), lambda i, k: (i, k)),
        pl.BlockSpec((tk, 2 * hacc), lambda i, k: (k, 0)),
        pl.BlockSpec((tm, h2), lambda i, k: (i, 0)),
    ]
    scratch = [pltpu.VMEM((tm, hacc), jnp.float32),
               pltpu.VMEM((tm, hacc), jnp.float32)]
    params = pltpu.CompilerParams(
        dimension_semantics=("parallel", "arbitrary"),
        vmem_limit_bytes=56 * 1024 * 1024,
    )
    if deep:
        in_specs += [pl.BlockSpec((tm, 1), lambda i, k: (i, 0)),
                     pl.BlockSpec((tm, 1), lambda i, k: (i, 0))]
        return pl.pallas_call(
            _sage_deep_kernel,
            out_shape=(jax.ShapeDtypeStruct((n, h2), jnp.float32),
                       jax.ShapeDtypeStruct((n, h2), jnp.bfloat16)),
            grid_spec=pltpu.PrefetchScalarGridSpec(
                num_scalar_prefetch=0, grid=grid, in_specs=in_specs,
                out_specs=(pl.BlockSpec((tm, h2), lambda i, k: (i, 0)),
                           pl.BlockSpec((tm, h2), lambda i, k: (i, 0))),
                scratch_shapes=scratch),
            compiler_params=params,
        )(cp, cn, y, s, sp, sn)
    return pl.pallas_call(
        _sage_base_kernel,
        out_shape=(jax.ShapeDtypeStruct((n, h2), jnp.bfloat16),
                   jax.ShapeDtypeStruct((n, 1), jnp.float32),
                   jax.ShapeDtypeStruct((n, 1), jnp.float32)),
        grid_spec=pltpu.PrefetchScalarGridSpec(
            num_scalar_prefetch=0, grid=grid, in_specs=in_specs,
            out_specs=(pl.BlockSpec((tm, h2), lambda i, k: (i, 0)),
                       pl.BlockSpec((tm, 1), lambda i, k: (i, 0)),
                       pl.BlockSpec((tm, 1), lambda i, k: (i, 0))),
            scratch_shapes=scratch),
        compiler_params=params,
    )(cp, cn, y, s)


def _loss_kernel(pzi_ref, pzj_ref, pzk_ref, nzi_ref, nzj_ref, nzk_ref,
                 t0_ref, t1_ref, t2_ref, t3_ref, t4_ref, t5_ref,
                 wab_ref, wfc_ref,
                 hp_ref, hn_ref, nl_ref,
                 p0_ref, p1_ref, p2_ref, p3_ref, p4_ref, p5_ref,
                 hps_ref, hns_ref, nls_ref, *, n_pos, n_neg, eb):
    c = pl.program_id(0)
    e = pl.program_id(1)

    @pl.when(e == 0)
    def _():
        hps_ref[...] = jnp.zeros_like(hps_ref)
        hns_ref[...] = jnp.zeros_like(hns_ref)
        nls_ref[...] = jnp.zeros_like(nls_ref)

    te = pzi_ref.shape[0]
    rows = (c * eb + e) * te + jax.lax.broadcasted_iota(jnp.int32, (te, 1), 0)
    pmask = rows < n_pos
    nmask = rows < n_neg

    blocks = [pzi_ref[...], pzj_ref[...], pzk_ref[...],
              nzi_ref[...], nzj_ref[...], nzk_ref[...]]
    pzi, pzj, pzk, nzi, nzj, nzk = [b.astype(jnp.float32) for b in blocks]

    d_pij = jnp.sum((pzi - pzj) ** 2, axis=-1, keepdims=True)
    d_pik = jnp.sum((pzi - pzk) ** 2, axis=-1, keepdims=True)
    hps_ref[...] += jnp.sum(jnp.where(pmask, jnp.maximum(d_pij - d_pik, 0.0), 0.0))
    d_nij = jnp.sum((nzi - nzj) ** 2, axis=-1, keepdims=True)
    d_nik = jnp.sum((nzi - nzk) ** 2, axis=-1, keepdims=True)
    hns_ref[...] += jnp.sum(jnp.where(nmask, jnp.maximum(d_nik - d_nij, 0.0), 0.0))

    stacked = jnp.concatenate(blocks, axis=0)
    proj = jnp.dot(stacked, wab_ref[...], preferred_element_type=jnp.float32)
    h = proj.shape[1] // 2

    def a_of(g):
        return proj[g * te:(g + 1) * te, :h]

    def b_of(g):
        return proj[g * te:(g + 1) * te, h:]

    pairs = [(0, 1), (3, 4), (3, 5), (4, 5), (0, 2), (1, 2)]
    p_all = jnp.concatenate([a_of(a) + b_of(b) for a, b in pairs], axis=0)
    logits_all = jnp.dot(p_all, wfc_ref[...],
                         preferred_element_type=jnp.float32)

    tgts = [t0_ref[...], t1_ref[...], t2_ref[...], t3_ref[...],
            t4_ref[...], t5_ref[...]]
    masks = [pmask, nmask, nmask, nmask, pmask, pmask]
    preds = [p0_ref, p1_ref, p2_ref, p3_ref, p4_ref, p5_ref]
    nll_sum = 0.0
    for g in range(6):
        logits = logits_all[g * te:(g + 1) * te, :]
        mx = jnp.max(logits, axis=-1, keepdims=True)
        lse = mx + jnp.log(jnp.sum(jnp.exp(logits - mx), axis=-1, keepdims=True))
        logsm = logits - lse
        preds[g][...] = logsm
        classes = jax.lax.broadcasted_iota(jnp.int32, logsm.shape, 1)
        onehot = (classes == tgts[g]).astype(jnp.float32)
        nll = -jnp.sum(onehot * logsm, axis=-1, keepdims=True)
        nll_sum = nll_sum + jnp.sum(jnp.where(masks[g], nll, 0.0))
    nls_ref[...] += nll_sum

    @pl.when(e == eb - 1)
    def _():
        hp_ref[...] = hps_ref[...][None]
        hn_ref[...] = hns_ref[...][None]
        nl_ref[...] = nls_ref[...][None]


def _fused_loss_opt(pz_i, pz_j, pz_k, nz_i, nz_j, nz_k, w_reg, w_fc,
                    target, lamb):
    e_pos, two_h = pz_i.shape
    e_neg = nz_i.shape[0]
    m_rows = 3 * (e_pos + e_neg)
    e_max = max(e_pos, e_neg)
    te = _EDGE_TILE if e_max > _EDGE_TILE else max(_round_up(e_max, 8), 8)
    e_pad = _round_up(e_max, te)
    nblk = e_pad // te
    ncore = 2 if nblk % 2 == 0 else 1
    eb = nblk // ncore

    def pad_rows(a):
        return jnp.pad(a, ((0, e_pad - a.shape[0]), (0, 0)))

    zs = [pad_rows(a) for a in (pz_i, pz_j, pz_k, nz_i, nz_j, nz_k)]

    sizes = [e_pos, e_neg, e_neg, e_neg, e_pos, e_pos]
    offs = [0]
    for sz in sizes[:-1]:
        offs.append(offs[-1] + sz)
    tgt = target.astype(jnp.int32)
    if all(o % te == 0 for o in offs) and all(sz == e_pad for sz in sizes):
        tgts = [tgt.reshape(-1, 1)] * 6
        tgt_specs = [pl.BlockSpec((te, 1),
                                  lambda c, e, og=o // te, eb=eb: (og + c * eb + e, 0))
                     for o in offs]
    else:
        tgts = [jnp.pad(tgt[o:o + sz], (0, e_pad - sz)).reshape(e_pad, 1)
                for o, sz in zip(offs, sizes)]
        tgt_specs = None

    h = w_reg.shape[1]
    wab = jnp.concatenate([w_reg[:two_h], w_reg[two_h:]],
                          axis=1).astype(jnp.bfloat16)

    kern = functools.partial(_loss_kernel, n_pos=e_pos, n_neg=e_neg, eb=eb)
    edge_spec = pl.BlockSpec((te, two_h), lambda c, e, eb=eb: (c * eb + e, 0))
    if tgt_specs is None:
        tgt_specs = [pl.BlockSpec((te, 1),
                                  lambda c, e, eb=eb: (c * eb + e, 0))] * 6
    pred_spec = pl.BlockSpec((te, 3), lambda c, e, eb=eb: (c * eb + e, 0))
    part_spec = pl.BlockSpec((1, 1, 1), lambda c, e: (c, 0, 0))

    outs = pl.pallas_call(
        kern,
        out_shape=(tuple(jax.ShapeDtypeStruct((ncore, 1, 1), jnp.float32)
                         for _ in range(3))
                   + tuple(jax.ShapeDtypeStruct((e_pad, 3), jnp.float32)
                           for _ in range(6))),
        grid_spec=pltpu.PrefetchScalarGridSpec(
            num_scalar_prefetch=0,
            grid=(ncore, eb),
            in_specs=[edge_spec] * 6 + tgt_specs
                     + [pl.BlockSpec((two_h, two_h), lambda c, e: (0, 0)),
                        pl.BlockSpec((h, 3), lambda c, e: (0, 0))],
            out_specs=(part_spec,) * 3 + (pred_spec,) * 6,
            scratch_shapes=[pltpu.VMEM((1, 1), jnp.float32)] * 3),
        compiler_params=pltpu.CompilerParams(
            dimension_semantics=("parallel", "arbitrary")),
    )(*zs, *tgts, wab, w_fc)

    hp_sum = jnp.sum(outs[0])
    hn_sum = jnp.sum(outs[1])
    nl_sum = jnp.sum(outs[2])
    loss = (nl_sum * (1.0 / m_rows)
            + lamb * (hp_sum * (1.0 / e_pos) + hn_sum * (1.0 / e_neg)))
    preds = outs[3:]
    predictions_soft = jnp.concatenate(
        [p[:sz] for p, sz in zip(preds, sizes)], axis=0)
    return loss, predictions_soft


def _adj_counts(edges, n):
    row, col = edges[0], edges[1]
    w = row != col
    return jnp.zeros((n, n), jnp.int8).at[row, col].add(w.astype(jnp.int8))


def kernel(X, pos_edges, neg_edges, target, pos_surr, neg_surr,
           pos_base_w, pos_base_b, neg_base_w, neg_base_b,
           pos_deep_w, pos_deep_b, neg_deep_w, neg_deep_b,
           regression_weights, fc_weights):
    n, f = X.shape
    h1 = pos_base_w.shape[1]
    h2d = pos_deep_w.shape[1]

    cp = _adj_counts(pos_edges, n)
    cn = _adj_counts(neg_edges, n)

    wy1 = jnp.concatenate([pos_base_w[:f], neg_base_w[:f]], axis=1)
    ws1 = jnp.concatenate([pos_base_w[f:], neg_base_w[f:]], axis=1)
    b1 = jnp.concatenate([pos_base_b, neg_base_b], axis=1)
    y1, s1 = _project(X, wy1, ws1, b1, ones_split=True)
    hc, dp, dn = _sage_layer(cp, cn, y1, s1, None, None, deep=False)
    sp2 = 1.0 / (dp + 1.0)
    sn2 = 1.0 / (dn + 1.0)

    wp, wn = pos_deep_w, neg_deep_w
    z128 = jnp.zeros((h2d, h2d), jnp.float32)
    w1d = jnp.concatenate(
        [jnp.concatenate([wp[0:h1], z128], axis=1),
         jnp.concatenate([z128, wn[0:h1]], axis=1)], axis=0)
    w2d = jnp.concatenate(
        [jnp.concatenate([z128, wn[h1:2 * h1]], axis=1),
         jnp.concatenate([wp[h1:2 * h1], z128], axis=1)], axis=0)
    w3d = jnp.concatenate(
        [jnp.concatenate([wp[2 * h1:3 * h1], wn[3 * h1:4 * h1]], axis=1),
         jnp.concatenate([wp[3 * h1:4 * h1], wn[2 * h1:3 * h1]], axis=1)],
        axis=0)
    b2 = jnp.concatenate([pos_deep_b, neg_deep_b], axis=1)
    y2, s2 = _project(hc, jnp.concatenate([w1d, w2d], axis=1), w3d, b2)
    z, zb = _sage_layer(cp, cn, y2, s2, sp2, sn2, deep=True)

    pz_i = jnp.take(zb, pos_edges[0], axis=0)
    pz_j = jnp.take(zb, pos_edges[1], axis=0)
    pz_k = jnp.take(zb, pos_surr, axis=0)
    nz_i = jnp.take(zb, neg_edges[0], axis=0)
    nz_j = jnp.take(zb, neg_edges[1], axis=0)
    nz_k = jnp.take(zb, neg_surr, axis=0)

    loss, predictions_soft = _fused_loss_opt(
        pz_i, pz_j, pz_k, nz_i, nz_j, nz_k,
        regression_weights, fc_weights, target, 1.0)
    return loss, z, predictions_soft


# speedup vs baseline: 1.4521x; 1.3852x over previous
import functools

import jax
import jax.numpy as jnp
from jax.experimental import pallas as pl
from jax.experimental.pallas import tpu as pltpu

_ROW_TILE = 2048
_EDGE_TILE = 512


def _round_up(x, m):
    return ((x + m - 1) // m) * m


def _row_tile(n, cap=_ROW_TILE):
    t = min(cap, n)
    while t > 128 and n % t:
        t -= 128
    return t


def _proj_kernel(x_ref, wy_ref, ws_ref, b_ref, y_ref, s_ref, *, ones_split):
    x = x_ref[...].astype(jnp.bfloat16)
    y = jnp.dot(x, wy_ref[...], preferred_element_type=jnp.float32)
    if ones_split:
        h = y.shape[1] // 2
        lane = jax.lax.broadcasted_iota(jnp.int32, (y.shape[0], h), 1)
        e = (lane == 0).astype(jnp.bfloat16)
        yb = y.astype(jnp.bfloat16)
        y_ref[...] = jnp.concatenate([yb[:, :h], e, yb[:, h:], e], axis=1)
    else:
        y_ref[...] = y.astype(jnp.bfloat16)
    s_ref[...] = (jnp.dot(x, ws_ref[...], preferred_element_type=jnp.float32)
                  + b_ref[...])


def _project(x, wy, ws, b, ones_split=False):
    n, f = x.shape
    tm = _row_tile(n, 1024)
    wy = wy.astype(jnp.bfloat16)
    ws = ws.astype(jnp.bfloat16)
    y_cols = 2 * wy.shape[1] if ones_split else wy.shape[1]
    return pl.pallas_call(
        functools.partial(_proj_kernel, ones_split=ones_split),
        out_shape=(jax.ShapeDtypeStruct((n, y_cols), jnp.bfloat16),
                   jax.ShapeDtypeStruct((n, ws.shape[1]), jnp.float32)),
        grid=(n // tm,),
        in_specs=[
            pl.BlockSpec((tm, f), lambda i: (i, 0)),
            pl.BlockSpec(wy.shape, lambda i: (0, 0)),
            pl.BlockSpec(ws.shape, lambda i: (0, 0)),
            pl.BlockSpec(b.shape, lambda i: (0, 0)),
        ],
        out_specs=(pl.BlockSpec((tm, y_cols), lambda i: (i, 0)),
                   pl.BlockSpec((tm, ws.shape[1]), lambda i: (i, 0))),
        compiler_params=pltpu.CompilerParams(
            dimension_semantics=("parallel",)),
    )(x, wy, ws, b)


def _norm_tanh(hp, hn, dtype):
    inv_p = jax.lax.rsqrt(
        jnp.maximum(jnp.sum(hp * hp, axis=-1, keepdims=True), 1e-24))
    inv_n = jax.lax.rsqrt(
        jnp.maximum(jnp.sum(hn * hn, axis=-1, keepdims=True), 1e-24))
    return jnp.tanh(
        jnp.concatenate([hp * inv_p, hn * inv_n], axis=-1)).astype(dtype)


def _sage_base_kernel(cp_ref, cn_ref, y_ref, s_ref,
                      o_ref, dp_ref, dn_ref, accp_ref, accn_ref):
    k = pl.program_id(1)

    @pl.when(k == 0)
    def _():
        accp_ref[...] = jnp.zeros_like(accp_ref)
        accn_ref[...] = jnp.zeros_like(accn_ref)

    y = y_ref[...]
    w = accp_ref.shape[1]
    accp_ref[...] += jnp.dot(cp_ref[...].astype(jnp.bfloat16), y[:, :w],
                             preferred_element_type=jnp.float32)
    accn_ref[...] += jnp.dot(cn_ref[...].astype(jnp.bfloat16), y[:, w:],
                             preferred_element_type=jnp.float32)

    @pl.when(k == pl.num_programs(1) - 1)
    def _():
        h = w // 2
        s = s_ref[...]
        accp = accp_ref[...]
        accn = accn_ref[...]
        dp = accp[:, h:h + 1]
        dn = accn[:, h:h + 1]
        sp = jnp.where(dp > 0, 1.0 / jnp.maximum(dp, 1.0), 0.0)
        sn = jnp.where(dn > 0, 1.0 / jnp.maximum(dn, 1.0), 0.0)
        hp = accp[:, :h] * sp + s[:, :h]
        hn = accn[:, :h] * sn + s[:, h:]
        o_ref[...] = _norm_tanh(hp, hn, o_ref.dtype)
        dp_ref[...] = dp
        dn_ref[...] = dn


def _sage_deep_kernel(cp_ref, cn_ref, y_ref, s_ref, sp_ref, sn_ref,
                      o32_ref, o3_ref, accp_ref, accn_ref):
    i = pl.program_id(0)
    k = pl.program_id(1)

    @pl.when(k == 0)
    def _():
        accp_ref[...] = jnp.zeros_like(accp_ref)
        accn_ref[...] = jnp.zeros_like(accn_ref)

    y = y_ref[...]
    h2 = accp_ref.shape[1]
    y1, y2 = y[:, :h2], y[:, h2:]
    accp_ref[...] += jnp.dot(cp_ref[...].astype(jnp.bfloat16), y1,
                             preferred_element_type=jnp.float32)
    accn_ref[...] += jnp.dot(cn_ref[...].astype(jnp.bfloat16), y2,
                             preferred_element_type=jnp.float32)

    @pl.when(k == i)
    def _():
        accp_ref[...] += y1.astype(jnp.float32)
        accn_ref[...] += y2.astype(jnp.float32)

    @pl.when(k == pl.num_programs(1) - 1)
    def _():
        h = (accp_ref[...] * sp_ref[...] + accn_ref[...] * sn_ref[...]
             + s_ref[...])
        hh = h.shape[1] // 2
        z = _norm_tanh(h[:, :hh], h[:, hh:], jnp.float32)
        o32_ref[...] = z
        o3_ref[...] = z[:, None, :]


def _sage_layer(cp, cn, y, s, sp, sn, deep):
    n = cp.shape[0]
    tm = tk = _row_tile(n)
    h2 = s.shape[1]
    hacc = y.shape[1] // 2
    grid = (n // tm, n // tk)
    in_specs = [
        pl.BlockSpec((tm, tk), lambda i, k: (i, k)),
        pl.BlockSpec((tm, tk), lambda i, k: (i, k)),
        pl.BlockSpec((tk, 2 * hacc), lambda i, k: (k, 0)),
        pl.BlockSpec((tm, h2), lambda i, k: (i, 0)),
    ]
    scratch = [pltpu.VMEM((tm, hacc), jnp.float32),
               pltpu.VMEM((tm, hacc), jnp.float32)]
    params = pltpu.CompilerParams(
        dimension_semantics=("parallel", "arbitrary"),
        vmem_limit_bytes=56 * 1024 * 1024,
    )
    if deep:
        in_specs += [pl.BlockSpec((tm, 1), lambda i, k: (i, 0)),
                     pl.BlockSpec((tm, 1), lambda i, k: (i, 0))]
        return pl.pallas_call(
            _sage_deep_kernel,
            out_shape=(jax.ShapeDtypeStruct((n, h2), jnp.float32),
                       jax.ShapeDtypeStruct((n, 1, h2), jnp.float32)),
            grid_spec=pltpu.PrefetchScalarGridSpec(
                num_scalar_prefetch=0, grid=grid, in_specs=in_specs,
                out_specs=(pl.BlockSpec((tm, h2), lambda i, k: (i, 0)),
                           pl.BlockSpec((tm, 1, h2), lambda i, k: (i, 0, 0))),
                scratch_shapes=scratch),
            compiler_params=params,
        )(cp, cn, y, s, sp, sn)
    return pl.pallas_call(
        _sage_base_kernel,
        out_shape=(jax.ShapeDtypeStruct((n, h2), jnp.bfloat16),
                   jax.ShapeDtypeStruct((n, 1), jnp.float32),
                   jax.ShapeDtypeStruct((n, 1), jnp.float32)),
        grid_spec=pltpu.PrefetchScalarGridSpec(
            num_scalar_prefetch=0, grid=grid, in_specs=in_specs,
            out_specs=(pl.BlockSpec((tm, h2), lambda i, k: (i, 0)),
                       pl.BlockSpec((tm, 1), lambda i, k: (i, 0)),
                       pl.BlockSpec((tm, 1), lambda i, k: (i, 0))),
            scratch_shapes=scratch),
        compiler_params=params,
    )(cp, cn, y, s)


def _loss_kernel(z3_ref,
                 i0_ref, i1_ref, i2_ref, i3_ref, i4_ref, i5_ref,
                 t0_ref, t1_ref, t2_ref, t3_ref, t4_ref, t5_ref,
                 wab_ref, wfc_ref,
                 hp_ref, hn_ref, nl_ref,
                 p0_ref, p1_ref, p2_ref, p3_ref, p4_ref, p5_ref,
                 g6_ref, hps_ref, hns_ref, nls_ref, *, n_pos, n_neg, eb):
    c = pl.program_id(0)
    e = pl.program_id(1)

    @pl.when(e == 0)
    def _():
        hps_ref[...] = jnp.zeros_like(hps_ref)
        hns_ref[...] = jnp.zeros_like(hns_ref)
        nls_ref[...] = jnp.zeros_like(nls_ref)

    te = t0_ref.shape[0]
    rows = (c * eb + e) * te + jax.lax.broadcasted_iota(jnp.int32, (te, 1), 0)
    pmask = rows < n_pos
    nmask = rows < n_neg

    for g, idx_ref in enumerate((i0_ref, i1_ref, i2_ref, i3_ref, i4_ref,
                                 i5_ref)):
        base = g * te

        def chunk(o, _, idx_ref=idx_ref, base=base):
            for sub in range(4):
                r0 = o * 32 + sub * 8
                rows8 = [z3_ref[idx_ref[r0 + u]] for u in range(8)]
                blk = jnp.concatenate(rows8, axis=0)
                g6_ref[pl.ds(pl.multiple_of(base + r0, 8), 8), :] = blk
            return 0

        jax.lax.fori_loop(0, te // 32, chunk, 0, unroll=False)

    gz = g6_ref[...]
    pzi, pzj, pzk, nzi, nzj, nzk = [gz[g * te:(g + 1) * te] for g in range(6)]

    d_pij = jnp.sum((pzi - pzj) ** 2, axis=-1, keepdims=True)
    d_pik = jnp.sum((pzi - pzk) ** 2, axis=-1, keepdims=True)
    hps_ref[...] += jnp.sum(jnp.where(pmask, jnp.maximum(d_pij - d_pik, 0.0), 0.0))
    d_nij = jnp.sum((nzi - nzj) ** 2, axis=-1, keepdims=True)
    d_nik = jnp.sum((nzi - nzk) ** 2, axis=-1, keepdims=True)
    hns_ref[...] += jnp.sum(jnp.where(nmask, jnp.maximum(d_nik - d_nij, 0.0), 0.0))

    proj = jnp.dot(gz.astype(jnp.bfloat16), wab_ref[...],
                   preferred_element_type=jnp.float32)
    h = proj.shape[1] // 2

    def a_of(g):
        return proj[g * te:(g + 1) * te, :h]

    def b_of(g):
        return proj[g * te:(g + 1) * te, h:]

    pairs = [(0, 1), (3, 4), (3, 5), (4, 5), (0, 2), (1, 2)]
    p_all = jnp.concatenate([a_of(a) + b_of(b) for a, b in pairs], axis=0)
    logits_all = jnp.dot(p_all, wfc_ref[...],
                         preferred_element_type=jnp.float32)

    tgts = [t0_ref[...], t1_ref[...], t2_ref[...], t3_ref[...],
            t4_ref[...], t5_ref[...]]
    masks = [pmask, nmask, nmask, nmask, pmask, pmask]
    preds = [p0_ref, p1_ref, p2_ref, p3_ref, p4_ref, p5_ref]
    nll_sum = 0.0
    for g in range(6):
        logits = logits_all[g * te:(g + 1) * te, :]
        mx = jnp.max(logits, axis=-1, keepdims=True)
        lse = mx + jnp.log(jnp.sum(jnp.exp(logits - mx), axis=-1, keepdims=True))
        logsm = logits - lse
        preds[g][...] = logsm
        classes = jax.lax.broadcasted_iota(jnp.int32, logsm.shape, 1)
        onehot = (classes == tgts[g]).astype(jnp.float32)
        nll = -jnp.sum(onehot * logsm, axis=-1, keepdims=True)
        nll_sum = nll_sum + jnp.sum(jnp.where(masks[g], nll, 0.0))
    nls_ref[...] += nll_sum

    @pl.when(e == eb - 1)
    def _():
        hp_ref[...] = hps_ref[...][None]
        hn_ref[...] = hns_ref[...][None]
        nl_ref[...] = nls_ref[...][None]


def _fused_loss_opt(z3, idxs, w_reg, w_fc, target, lamb):
    e_pos = idxs[0].shape[0]
    e_neg = idxs[3].shape[0]
    two_h = z3.shape[2]
    m_rows = 3 * (e_pos + e_neg)
    e_max = max(e_pos, e_neg)
    te = _EDGE_TILE if e_max > _EDGE_TILE else max(_round_up(e_max, 32), 32)
    e_pad = _round_up(e_max, te)
    nblk = e_pad // te
    ncore = 2 if nblk % 2 == 0 else 1
    eb = nblk // ncore

    idxs = [jnp.pad(a.astype(jnp.int32), (0, e_pad - a.shape[0]))
            for a in idxs]

    sizes = [e_pos, e_neg, e_neg, e_neg, e_pos, e_pos]
    offs = [0]
    for sz in sizes[:-1]:
        offs.append(offs[-1] + sz)
    tgt = target.astype(jnp.int32)
    if all(o % te == 0 for o in offs) and all(sz == e_pad for sz in sizes):
        tgts = [tgt.reshape(-1, 1)] * 6
        tgt_specs = [pl.BlockSpec((te, 1),
                                  lambda c, e, og=o // te, eb=eb: (og + c * eb + e, 0))
                     for o in offs]
    else:
        tgts = [jnp.pad(tgt[o:o + sz], (0, e_pad - sz)).reshape(e_pad, 1)
                for o, sz in zip(offs, sizes)]
        tgt_specs = None

    h = w_reg.shape[1]
    wab = jnp.concatenate([w_reg[:two_h], w_reg[two_h:]],
                          axis=1).astype(jnp.bfloat16)

    kern = functools.partial(_loss_kernel, n_pos=e_pos, n_neg=e_neg, eb=eb)
    z3_spec = pl.BlockSpec(z3.shape, lambda c, e: (0, 0, 0))
    idx_spec = pl.BlockSpec((te,), lambda c, e, eb=eb: (c * eb + e,),
                            memory_space=pltpu.SMEM)
    if tgt_specs is None:
        tgt_specs = [pl.BlockSpec((te, 1),
                                  lambda c, e, eb=eb: (c * eb + e, 0))] * 6
    pred_spec = pl.BlockSpec((te, 3), lambda c, e, eb=eb: (c * eb + e, 0))
    part_spec = pl.BlockSpec((1, 1, 1), lambda c, e: (c, 0, 0))

    outs = pl.pallas_call(
        kern,
        out_shape=(tuple(jax.ShapeDtypeStruct((ncore, 1, 1), jnp.float32)
                         for _ in range(3))
                   + tuple(jax.ShapeDtypeStruct((e_pad, 3), jnp.float32)
                           for _ in range(6))),
        grid_spec=pltpu.PrefetchScalarGridSpec(
            num_scalar_prefetch=0,
            grid=(ncore, eb),
            in_specs=[z3_spec] + [idx_spec] * 6 + tgt_specs
                     + [pl.BlockSpec((two_h, two_h), lambda c, e: (0, 0)),
                        pl.BlockSpec((h, 3), lambda c, e: (0, 0))],
            out_specs=(part_spec,) * 3 + (pred_spec,) * 6,
            scratch_shapes=[pltpu.VMEM((6 * te, two_h), jnp.float32)]
                           + [pltpu.VMEM((1, 1), jnp.float32)] * 3),
        compiler_params=pltpu.CompilerParams(
            dimension_semantics=("parallel", "arbitrary"),
            vmem_limit_bytes=56 * 1024 * 1024),
    )(z3, *idxs, *tgts, wab, w_fc)

    hp_sum = jnp.sum(outs[0])
    hn_sum = jnp.sum(outs[1])
    nl_sum = jnp.sum(outs[2])
    loss = (nl_sum * (1.0 / m_rows)
            + lamb * (hp_sum * (1.0 / e_pos) + hn_sum * (1.0 / e_neg)))
    preds = outs[3:]
    predictions_soft = jnp.concatenate(
        [p[:sz] for p, sz in zip(preds, sizes)], axis=0)
    return loss, predictions_soft


def _adj_counts(edges, n):
    row, col = edges[0], edges[1]
    w = row != col
    return jnp.zeros((n, n), jnp.int8).at[row, col].add(w.astype(jnp.int8))


def kernel(X, pos_edges, neg_edges, target, pos_surr, neg_surr,
           pos_base_w, pos_base_b, neg_base_w, neg_base_b,
           pos_deep_w, pos_deep_b, neg_deep_w, neg_deep_b,
           regression_weights, fc_weights):
    n, f = X.shape
    h1 = pos_base_w.shape[1]
    h2d = pos_deep_w.shape[1]

    cp = _adj_counts(pos_edges, n)
    cn = _adj_counts(neg_edges, n)

    wy1 = jnp.concatenate([pos_base_w[:f], neg_base_w[:f]], axis=1)
    ws1 = jnp.concatenate([pos_base_w[f:], neg_base_w[f:]], axis=1)
    b1 = jnp.concatenate([pos_base_b, neg_base_b], axis=1)
    y1, s1 = _project(X, wy1, ws1, b1, ones_split=True)
    hc, dp, dn = _sage_layer(cp, cn, y1, s1, None, None, deep=False)
    sp2 = 1.0 / (dp + 1.0)
    sn2 = 1.0 / (dn + 1.0)

    wp, wn = pos_deep_w, neg_deep_w
    z128 = jnp.zeros((h2d, h2d), jnp.float32)
    w1d = jnp.concatenate(
        [jnp.concatenate([wp[0:h1], z128], axis=1),
         jnp.concatenate([z128, wn[0:h1]], axis=1)], axis=0)
    w2d = jnp.concatenate(
        [jnp.concatenate([z128, wn[h1:2 * h1]], axis=1),
         jnp.concatenate([wp[h1:2 * h1], z128], axis=1)], axis=0)
    w3d = jnp.concatenate(
        [jnp.concatenate([wp[2 * h1:3 * h1], wn[3 * h1:4 * h1]], axis=1),
         jnp.concatenate([wp[3 * h1:4 * h1], wn[2 * h1:3 * h1]], axis=1)],
        axis=0)
    b2 = jnp.concatenate([pos_deep_b, neg_deep_b], axis=1)
    y2, s2 = _project(hc, jnp.concatenate([w1d, w2d], axis=1), w3d, b2)
    z, z3 = _sage_layer(cp, cn, y2, s2, sp2, sn2, deep=True)

    idxs = (pos_edges[0], pos_edges[1], pos_surr,
            neg_edges[0], neg_edges[1], neg_surr)
    loss, predictions_soft = _fused_loss_opt(
        z3, idxs, regression_weights, fc_weights, target, 1.0)
    return loss, z, predictions_soft
